# Initial kernel scaffold; baseline (speedup 1.0000x reference)
import jax
import jax.numpy as jnp
from jax.experimental import pallas as pl


def kernel(x, conv_w, conv_b, bn_w, bn_b):
    raise NotImplementedError("write your pallas kernel here")


# speedup vs baseline: 21.9271x; 21.9271x over previous
import functools
import math

import jax
import jax.numpy as jnp
from jax.experimental import pallas as pl

B, C, H, W = 2, 96, 56, 56
K = 9
R = 2
GROUPS = 4
N = H * W
M = (H // R) * (W // R)
BN = 784
NBLK = (B * N) // BN

_BIG = 3.0e38


def _fused_body(xb_ref, ycm_ref, ymc_ref, wd_ref, wb_ref, beta_ref, out_ref):
    blk = pl.program_id(0)
    xb = xb_ref[...]
    ycm = ycm_ref[0]
    ymc = ymc_ref[0]

    xnorm = jnp.sqrt(jnp.sum(xb * xb, axis=1, keepdims=True))
    xn = xb / jnp.maximum(xnorm, 1e-12)
    ynorm = jnp.sqrt(jnp.sum(ycm * ycm, axis=0, keepdims=True))
    yn = ycm / jnp.maximum(ynorm, 1e-12)
    q = jnp.sum(yn * yn, axis=0, keepdims=True)

    dist = q - 2.0 * jnp.dot(xn, yn, preferred_element_type=jnp.float32)

    lane = jax.lax.broadcasted_iota(jnp.int32, (BN, M), 1)
    acc = jnp.full((BN, C), -_BIG, jnp.float32)
    for _ in range(K):
        v = jnp.min(dist, axis=1, keepdims=True)
        idx = jnp.min(jnp.where(dist <= v, lane, M), axis=1, keepdims=True)
        sel = lane == idx
        onehot = sel.astype(jnp.float32)
        g = jnp.dot(onehot, ymc, preferred_element_type=jnp.float32)
        acc = jnp.maximum(acc, g)
        dist = jnp.where(sel, _BIG, dist)

    o = (jnp.dot(xb, wd_ref[...], preferred_element_type=jnp.float32)
         + jnp.dot(acc, wb_ref[...], preferred_element_type=jnp.float32)
         + beta_ref[0:1, :])
    out_ref[...] = o * 0.5 * (1.0 + jax.lax.erf(o * (1.0 / math.sqrt(2.0))))


@functools.partial(jax.jit, static_argnums=())
def kernel(x, conv_w, conv_b, bn_w, bn_b):
    xf = x.reshape(B, C, N)
    x_nc = xf.transpose(0, 2, 1).reshape(B * N, C)
    y = x.reshape(B, C, H // R, R, W // R, R).mean(axis=(3, 5))
    y_cm = y.reshape(B, C, M)
    y_mc = y_cm.transpose(0, 2, 1)

    w2 = conv_w[:, :, 0, 0]
    gout = C // GROUPS
    wa = jnp.zeros((C, C), jnp.float32)
    wb = jnp.zeros((C, C), jnp.float32)
    for g in range(GROUPS):
        sl = slice(g * gout, (g + 1) * gout)
        blk = w2[sl, :]
        wa = wa.at[sl, sl].set(blk[:, 0::2].T)
        wb = wb.at[sl, sl].set(blk[:, 1::2].T)
    alpha = bn_w * (1.0 / math.sqrt(1.0 + 1e-5))
    wa = wa * alpha[None, :]
    wb = wb * alpha[None, :]
    wd = wa - wb
    beta = conv_b * alpha + bn_b
    beta8 = jnp.broadcast_to(beta[None, :], (8, C))

    out_flat = pl.pallas_call(
        _fused_body,
        grid=(NBLK,),
        in_specs=[
            pl.BlockSpec((BN, C), lambda i: (i, 0)),
            pl.BlockSpec((1, C, M), lambda i: (i // (N // BN), 0, 0)),
            pl.BlockSpec((1, M, C), lambda i: (i // (N // BN), 0, 0)),
            pl.BlockSpec((C, C), lambda i: (0, 0)),
            pl.BlockSpec((C, C), lambda i: (0, 0)),
            pl.BlockSpec((8, C), lambda i: (0, 0)),
        ],
        out_specs=pl.BlockSpec((BN, C), lambda i: (i, 0)),
        out_shape=jax.ShapeDtypeStruct((B * N, C), jnp.float32),
    )(x_nc, y_cm, y_mc, wd, wb, beta8)

    return out_flat.reshape(B, N, C).transpose(0, 2, 1).reshape(B, C, H, W)
